# Initial kernel scaffold; baseline (speedup 1.0000x reference)
import jax
import jax.numpy as jnp
from jax.experimental import pallas as pl


def kernel(x, edge_index, globf, W1, b1, W2, b2, Wf1, bf1, Wf2, bf2, Wo, bo):
    raise NotImplementedError("write your pallas kernel here")


# speedup vs baseline: 13.8222x; 13.8222x over previous
import jax
import jax.numpy as jnp
from jax import lax
from jax.experimental import pallas as pl
from jax.experimental.pallas import tpu as pltpu
from jax.experimental.pallas import tpu_sc as plsc

N = 100000
E = 1600000
NP = 100096
NPAD = NP - N
EP = 1638400
B = 2048
NSC = 2
NT = 16
RPT = NP // NT
DROW = 6272
DLAST = NP - 15 * DROW
CH = 8
NCH = 64 // CH

_mesh = lambda: plsc.VectorSubcoreMesh(core_axis_name="c", subcore_axis_name="s")
_sc_params = lambda: pltpu.CompilerParams(use_tc_tiling_on_sc=False)


def _fill1d(ref, n16, value):
    def body(i, _):
        ref[pl.ds(i * 16, 16)] = jnp.full((16,), value, jnp.float32)
        return 0
    lax.fori_loop(0, n16, body, 0)


def _deg_body(dst_h, z1_h, out0_h, out1_h, dstb_v, ones_v, accum):
    c = lax.axis_index("c")
    s = lax.axis_index("s")
    row0 = s * DROW
    _fill1d(ones_v, B // 16, 1.0)

    @pl.when(s < 15)
    def _():
        pltpu.sync_copy(z1_h.at[pl.ds(row0, DROW)], accum.at[pl.ds(row0, DROW)])

    @pl.when(s == 15)
    def _():
        pltpu.sync_copy(z1_h.at[pl.ds(row0, DLAST)], accum.at[pl.ds(row0, DLAST)])

    plsc.subcore_barrier()
    e0 = (c * NT + s) * (EP // (NSC * NT))
    nb = EP // (NSC * NT) // B

    def body(j, _):
        base = pl.multiple_of(e0 + j * B, 128)
        pltpu.sync_copy(dst_h.at[pl.ds(base, B)], dstb_v)
        pltpu.sync_copy(ones_v, accum.at[dstb_v], add=True)
        return 0

    lax.fori_loop(0, nb, body, 0)
    plsc.subcore_barrier()
    for cc, out_h in ((0, out0_h), (1, out1_h)):
        @pl.when(c == cc)
        def _(out_h=out_h):
            @pl.when(s < 15)
            def _():
                pltpu.sync_copy(accum.at[pl.ds(row0, DROW)],
                                out_h.at[pl.ds(row0, DROW)])

            @pl.when(s == 15)
            def _():
                pltpu.sync_copy(accum.at[pl.ds(row0, DLAST)],
                                out_h.at[pl.ds(row0, DLAST)])


def _sc_degree(dst, z1):
    return pl.kernel(
        _deg_body,
        out_type=(jax.ShapeDtypeStruct((NP,), jnp.float32),
                  jax.ShapeDtypeStruct((NP,), jnp.float32)),
        mesh=_mesh(),
        compiler_params=_sc_params(),
        scratch_types=[
            pltpu.VMEM((B,), jnp.int32),
            pltpu.VMEM((B,), jnp.float32),
            pltpu.VMEM_SHARED((NP,), jnp.float32),
        ],
    )(dst, z1)


def _agg_pass(src_h, dst_h, table_h, zc_h, out_h, idx_v, dstb_v, rows_v,
              accum, e0, n_edges):
    s = lax.axis_index("s")
    row0 = s * RPT
    pltpu.sync_copy(zc_h.at[pl.ds(row0, RPT)], accum.at[pl.ds(row0, RPT)])
    plsc.subcore_barrier()

    def body(j, _):
        base = pl.multiple_of(e0 + j * B, 128)
        pltpu.sync_copy(src_h.at[pl.ds(base, B)], idx_v)
        pltpu.sync_copy(dst_h.at[pl.ds(base, B)], dstb_v)
        pltpu.sync_copy(table_h.at[idx_v], rows_v)
        pltpu.sync_copy(rows_v, accum.at[dstb_v], add=True)
        return 0

    lax.fori_loop(0, n_edges // B, body, 0)
    plsc.subcore_barrier()
    pltpu.sync_copy(accum.at[pl.ds(row0, RPT)], out_h.at[pl.ds(row0, RPT)])


def _agg1_body(src_h, dst_h, t_h, zc_h, out0_h, out1_h, idx_v, dstb_v,
               rows_v, accum):
    c = lax.axis_index("c")
    s = lax.axis_index("s")
    ept = EP // (NSC * NT)
    e0 = (c * NT + s) * ept
    for cc, out_h in ((0, out0_h), (1, out1_h)):
        @pl.when(c == cc)
        def _(out_h=out_h):
            _agg_pass(src_h, dst_h, t_h, zc_h, out_h, idx_v, dstb_v,
                      rows_v, accum, e0, ept)


def _sc_agg1(src, dst, table, zc):
    return pl.kernel(
        _agg1_body,
        out_type=(jax.ShapeDtypeStruct((NP, CH), jnp.float32),
                  jax.ShapeDtypeStruct((NP, CH), jnp.float32)),
        mesh=_mesh(),
        compiler_params=_sc_params(),
        scratch_types=[
            pltpu.VMEM((B,), jnp.int32),
            pltpu.VMEM((B,), jnp.int32),
            pltpu.VMEM((B, CH), jnp.float32),
            pltpu.VMEM_SHARED((NP, CH), jnp.float32),
        ],
    )(src, dst, table, zc)


def _agg2_body(src_h, dst_h, *refs):
    tables = refs[:NCH]
    zc_h = refs[NCH]
    outs = refs[NCH + 1:2 * NCH + 1]
    idx_v, dstb_v, rows_v, accum = refs[2 * NCH + 1:]
    c = lax.axis_index("c")
    s = lax.axis_index("s")
    ept = EP // NT
    e0 = s * ept
    for q in range(NCH):
        @pl.when(q // (NCH // NSC) == c)
        def _(q=q):
            _agg_pass(src_h, dst_h, tables[q], zc_h, outs[q], idx_v, dstb_v,
                      rows_v, accum, e0, ept)


def _sc_agg2(src, dst, tables, zc):
    return pl.kernel(
        _agg2_body,
        out_type=tuple(jax.ShapeDtypeStruct((NP, CH), jnp.float32)
                       for _ in range(NCH)),
        mesh=_mesh(),
        compiler_params=_sc_params(),
        scratch_types=[
            pltpu.VMEM((B,), jnp.int32),
            pltpu.VMEM((B,), jnp.int32),
            pltpu.VMEM((B, CH), jnp.float32),
            pltpu.VMEM_SHARED((NP, CH), jnp.float32),
        ],
    )(src, dst, *tables, zc)


_BN = 256


def _tcA_body(dp0, dp1, x8, dinv_o, hs1_o):
    deg = dp0[...] + dp1[...] + 1.0
    dinv = lax.rsqrt(deg)
    dinv_o[...] = dinv
    hs1_o[...] = dinv * x8[...]


def _tcA(dp0, dp1, x8):
    return pl.pallas_call(
        _tcA_body,
        grid=(NP // _BN,),
        in_specs=[
            pl.BlockSpec((_BN, 1), lambda i: (i, 0)),
            pl.BlockSpec((_BN, 1), lambda i: (i, 0)),
            pl.BlockSpec((_BN, CH), lambda i: (i, 0)),
        ],
        out_specs=[
            pl.BlockSpec((_BN, 1), lambda i: (i, 0)),
            pl.BlockSpec((_BN, CH), lambda i: (i, 0)),
        ],
        out_shape=[
            jax.ShapeDtypeStruct((NP, 1), jnp.float32),
            jax.ShapeDtypeStruct((NP, CH), jnp.float32),
        ],
    )(dp0, dp1, x8)


def _tcB_body(u1a, u1b, hs1, dinv, W1, b1, *outs):
    agg = dinv[...] * (u1a[...] + u1b[...] + hs1[...])
    h1 = jax.nn.relu(
        jnp.dot(agg[:, :5], W1[...], preferred_element_type=jnp.float32)
        + b1[...])
    hs2 = dinv[...] * h1
    i = pl.program_id(0)
    rows = i * _BN + lax.broadcasted_iota(jnp.int32, (_BN, 1), 0)
    hs2 = jnp.where(rows < N, hs2, 0.0)
    for q in range(NCH):
        outs[q][...] = hs2[:, CH * q:CH * (q + 1)]


def _tcB(u1a, u1b, hs1, dinv, W1, b1):
    spec8 = pl.BlockSpec((_BN, CH), lambda i: (i, 0))
    return pl.pallas_call(
        _tcB_body,
        grid=(NP // _BN,),
        in_specs=[
            spec8, spec8, spec8,
            pl.BlockSpec((_BN, 1), lambda i: (i, 0)),
            pl.BlockSpec((5, 64), lambda i: (0, 0)),
            pl.BlockSpec((64,), lambda i: (0,)),
        ],
        out_specs=[spec8] * NCH,
        out_shape=[jax.ShapeDtypeStruct((NP, CH), jnp.float32)] * NCH,
    )(u1a, u1b, hs1, dinv, W1, b1)


def _tcC_body(*refs):
    us = refs[:NCH]
    hs = refs[NCH:2 * NCH]
    (dinv, gf, W2, b2, Wf1, bf1, Wf2, bf2, Wo, bo, out_o) = refs[2 * NCH:]
    d = dinv[...]
    acc = jnp.broadcast_to(b2[...], (_BN, 64))
    for q in range(NCH):
        aggq = d * (us[q][...] + hs[q][...])
        acc = acc + jnp.dot(aggq, W2[CH * q:CH * (q + 1), :],
                            preferred_element_type=jnp.float32)
    h2v = jax.nn.relu(acc)
    t = jax.nn.relu(
        jnp.dot(h2v, Wf1[:64, :], preferred_element_type=jnp.float32)
        + jnp.dot(gf[...], Wf1[64:67, :], preferred_element_type=jnp.float32)
        + bf1[...])
    t = jax.nn.relu(
        jnp.dot(t, Wf2[...], preferred_element_type=jnp.float32) + bf2[...])
    out_o[...] = (
        jnp.dot(t, Wo[...], preferred_element_type=jnp.float32) + bo[...])


def _tcC(u2s, hs2s, dinv, gf, W2, b2, Wf1, bf1, Wf2, bf2, Wo, bo):
    spec8 = pl.BlockSpec((_BN, CH), lambda i: (i, 0))
    return pl.pallas_call(
        _tcC_body,
        grid=(NP // _BN,),
        in_specs=(
            [spec8] * (2 * NCH)
            + [
                pl.BlockSpec((_BN, 1), lambda i: (i, 0)),
                pl.BlockSpec((_BN, 3), lambda i: (i, 0)),
                pl.BlockSpec((64, 64), lambda i: (0, 0)),
                pl.BlockSpec((64,), lambda i: (0,)),
                pl.BlockSpec((67, 64), lambda i: (0, 0)),
                pl.BlockSpec((64,), lambda i: (0,)),
                pl.BlockSpec((64, 64), lambda i: (0, 0)),
                pl.BlockSpec((64,), lambda i: (0,)),
                pl.BlockSpec((64, 30), lambda i: (0, 0)),
                pl.BlockSpec((30,), lambda i: (0,)),
            ]
        ),
        out_specs=pl.BlockSpec((_BN, 30), lambda i: (i, 0)),
        out_shape=jax.ShapeDtypeStruct((NP, 30), jnp.float32),
    )(*u2s, *hs2s, dinv, gf, W2, b2, Wf1, bf1, Wf2, bf2, Wo, bo)


def kernel(x, edge_index, globf, W1, b1, W2, b2, Wf1, bf1, Wf2, bf2, Wo, bo):
    ei = edge_index.astype(jnp.int32)
    pad_tgt = N + (jnp.arange(EP - E, dtype=jnp.int32) % NPAD)
    src = jnp.concatenate([ei[0], pad_tgt])
    dst = jnp.concatenate([ei[1], pad_tgt])
    z1 = jnp.zeros((NP,), jnp.float32)
    zc = jnp.zeros((NP, CH), jnp.float32)

    dg0, dg1 = _sc_degree(dst, z1)
    x8 = jnp.pad(x, ((0, NPAD), (0, CH - x.shape[1])))
    dinv, hs1 = _tcA(dg0.reshape(NP, 1), dg1.reshape(NP, 1), x8)

    u1a, u1b = _sc_agg1(src, dst, hs1, zc)
    hs2 = _tcB(u1a, u1b, hs1, dinv, W1, b1)

    u2 = _sc_agg2(src, dst, hs2, zc)
    gf = jnp.pad(globf, ((0, NPAD), (0, 0)))
    out = _tcC(u2, hs2, dinv, gf, W2, b2, Wf1, bf1, Wf2, bf2, Wo, bo)
    return out[:N]
